# initial kernel scaffold (unmeasured)
import jax
import jax.numpy as jnp
from jax import lax
from jax.experimental import pallas as pl
from jax.experimental.pallas import tpu as pltpu

N_DEV = 32
SQ = 1024
D = 1024
HQ = 8
DH = 128
CHUNK = SQ // N_DEV
SCALE = 0.08838834764831843
BLK = 64


def kernel(x, Wq, K_ext, V_ext, Wo):
    def body(x_ref, wq_ref, k_hbm, v_hbm, wo_ref, out_ref,
             kv_vmem, q_bf, ctx_bf, partial_f32, partial_bf,
             ag_src, rs_buf, ag_buf,
             local_sems, rs_send, rs_recv, ag_send, ag_recv):
        my = lax.axis_index("i")
        h0 = my * HQ

        ck = pltpu.make_async_copy(
            k_hbm.at[0, :, pl.ds(h0, HQ), :], kv_vmem.at[0], local_sems.at[0])
        cv = pltpu.make_async_copy(
            v_hbm.at[0, :, pl.ds(h0, HQ), :], kv_vmem.at[1], local_sems.at[1])
        ck.start()
        cv.start()

        xb = x_ref[0].astype(jnp.bfloat16)
        wqb = wq_ref[...].astype(jnp.bfloat16)
        q_bf[...] = jnp.dot(
            xb, wqb, preferred_element_type=jnp.float32).astype(jnp.bfloat16)

        ck.wait()
        cv.wait()

        qb_i = lax.broadcasted_iota(jnp.int32, (SQ, SQ), 0) // BLK
        kb_i = lax.broadcasted_iota(jnp.int32, (SQ, SQ), 1) // BLK
        mask = (qb_i == kb_i) | (kb_i == 0) | ((qb_i + kb_i) % 3 == 0)

        for h in range(HQ):
            qh = q_bf[:, h * DH:(h + 1) * DH]
            kh = kv_vmem[0, :, h, :].astype(jnp.bfloat16)
            vh = kv_vmem[1, :, h, :].astype(jnp.bfloat16)
            s = lax.dot_general(
                qh, kh, (((1,), (1,)), ((), ())),
                preferred_element_type=jnp.float32) * SCALE
            s = jnp.where(mask, s, -1e9)
            m = jnp.max(s, axis=1, keepdims=True)
            w = jnp.exp(s - m)
            w = w / jnp.sum(w, axis=1, keepdims=True)
            ctx = jnp.dot(
                w.astype(jnp.bfloat16), vh, preferred_element_type=jnp.float32)
            ctx_bf[:, h * DH:(h + 1) * DH] = ctx.astype(jnp.bfloat16)

        wob = wo_ref[...].astype(jnp.bfloat16)
        part = jnp.dot(ctx_bf[...], wob, preferred_element_type=jnp.float32)
        partial_f32[...] = part
        partial_bf[...] = part.astype(jnp.bfloat16)

        rs_descs = []
        for d in range(1, N_DEV):
            p = (my + d) % N_DEV
            rdma = pltpu.make_async_remote_copy(
                src_ref=partial_bf.at[pl.ds(p * CHUNK, CHUNK), :],
                dst_ref=rs_buf.at[d - 1],
                send_sem=rs_send.at[d - 1],
                recv_sem=rs_recv.at[d - 1],
                device_id=(p,),
                device_id_type=pl.DeviceIdType.MESH,
            )
            rdma.start()
            rs_descs.append(rdma)

        acc = partial_f32[pl.ds(my * CHUNK, CHUNK), :]
        for d in range(1, N_DEV):
            recv = pltpu.make_async_remote_copy(
                src_ref=rs_buf.at[d - 1], dst_ref=rs_buf.at[d - 1],
                send_sem=rs_send.at[d - 1], recv_sem=rs_recv.at[d - 1],
                device_id=(my,), device_id_type=pl.DeviceIdType.MESH,
            )
            recv.wait_recv()
            acc = acc + rs_buf[d - 1].astype(jnp.float32)

        for r in rs_descs:
            r.wait_send()

        ag_src[...] = acc.astype(jnp.bfloat16)
        out_ref[0, pl.ds(my * CHUNK, CHUNK), :] = acc

        ag_descs = []
        for d in range(1, N_DEV):
            p = (my + d) % N_DEV
            rdma = pltpu.make_async_remote_copy(
                src_ref=ag_src,
                dst_ref=ag_buf.at[d - 1],
                send_sem=ag_send.at[d - 1],
                recv_sem=ag_recv.at[d - 1],
                device_id=(p,),
                device_id_type=pl.DeviceIdType.MESH,
            )
            rdma.start()
            ag_descs.append(rdma)

        for d in range(1, N_DEV):
            recv = pltpu.make_async_remote_copy(
                src_ref=ag_buf.at[d - 1], dst_ref=ag_buf.at[d - 1],
                send_sem=ag_send.at[d - 1], recv_sem=ag_recv.at[d - 1],
                device_id=(my,), device_id_type=pl.DeviceIdType.MESH,
            )
            recv.wait_recv()
            src = (my - d) % N_DEV
            out_ref[0, pl.ds(src * CHUNK, CHUNK), :] = (
                ag_buf[d - 1].astype(jnp.float32))

        for r in ag_descs:
            r.wait_send()

    return pl.pallas_call(
        body,
        out_shape=jax.ShapeDtypeStruct((1, SQ, D), jnp.float32),
        in_specs=[
            pl.BlockSpec(memory_space=pltpu.VMEM),
            pl.BlockSpec(memory_space=pltpu.VMEM),
            pl.BlockSpec(memory_space=pltpu.ANY),
            pl.BlockSpec(memory_space=pltpu.ANY),
            pl.BlockSpec(memory_space=pltpu.VMEM),
        ],
        out_specs=pl.BlockSpec(memory_space=pltpu.VMEM),
        scratch_shapes=[
            pltpu.VMEM((2, SQ, HQ, DH), jnp.float32),
            pltpu.VMEM((SQ, HQ * DH), jnp.bfloat16),
            pltpu.VMEM((SQ, HQ * DH), jnp.bfloat16),
            pltpu.VMEM((SQ, D), jnp.float32),
            pltpu.VMEM((SQ, D), jnp.bfloat16),
            pltpu.VMEM((CHUNK, D), jnp.bfloat16),
            pltpu.VMEM((N_DEV - 1, CHUNK, D), jnp.bfloat16),
            pltpu.VMEM((N_DEV - 1, CHUNK, D), jnp.bfloat16),
            pltpu.SemaphoreType.DMA((2,)),
            pltpu.SemaphoreType.DMA((N_DEV - 1,)),
            pltpu.SemaphoreType.DMA((N_DEV - 1,)),
            pltpu.SemaphoreType.DMA((N_DEV - 1,)),
            pltpu.SemaphoreType.DMA((N_DEV - 1,)),
        ],
    )(x, Wq, K_ext, V_ext, Wo)


# baseline (device time: 108382 ns/iter reference)
import jax
import jax.numpy as jnp
from jax import lax
from jax.experimental import pallas as pl
from jax.experimental.pallas import tpu as pltpu

N_DEV = 32
SQ = 1024
D = 1024
HQ = 8
DH = 128
CHUNK = SQ // N_DEV
SCALE = 0.08838834764831843
BLK = 64


def kernel(x, Wq, K_ext, V_ext, Wo):
    def body(x_ref, wq_ref, k_hbm, v_hbm, wo_ref, out_ref,
             kv_vmem, q_bf, ctx_bf, partial_f32, partial_bf,
             ag_src, rs_buf, ag_buf,
             local_sems, rs_send, rs_recv, ag_send, ag_recv):
        my = lax.axis_index("i")
        h0 = my * HQ

        ck = pltpu.make_async_copy(
            k_hbm.at[0, :, pl.ds(h0, HQ), :], kv_vmem.at[0], local_sems.at[0])
        cv = pltpu.make_async_copy(
            v_hbm.at[0, :, pl.ds(h0, HQ), :], kv_vmem.at[1], local_sems.at[1])
        ck.start()
        cv.start()

        xb = x_ref[0].astype(jnp.bfloat16)
        wqb = wq_ref[...].astype(jnp.bfloat16)
        q_bf[...] = jnp.dot(
            xb, wqb, preferred_element_type=jnp.float32).astype(jnp.bfloat16)

        ck.wait()
        cv.wait()

        qb_i = lax.broadcasted_iota(jnp.int32, (SQ, SQ), 0) // BLK
        kb_i = lax.broadcasted_iota(jnp.int32, (SQ, SQ), 1) // BLK
        mask = (qb_i == kb_i) | (kb_i == 0) | ((qb_i + kb_i) % 3 == 0)

        for h in range(HQ):
            qh = q_bf[:, h * DH:(h + 1) * DH]
            kh = kv_vmem[0, :, h, :].astype(jnp.bfloat16)
            vh = kv_vmem[1, :, h, :].astype(jnp.bfloat16)
            s = lax.dot_general(
                qh, kh, (((1,), (1,)), ((), ())),
                preferred_element_type=jnp.float32) * SCALE
            s = jnp.where(mask, s, -1e9)
            m = jnp.max(s, axis=1, keepdims=True)
            w = jnp.exp(s - m)
            w = w / jnp.sum(w, axis=1, keepdims=True)
            ctx = jnp.dot(
                w.astype(jnp.bfloat16), vh, preferred_element_type=jnp.float32)
            ctx_bf[:, h * DH:(h + 1) * DH] = ctx.astype(jnp.bfloat16)

        wob = wo_ref[...].astype(jnp.bfloat16)
        part = jnp.dot(ctx_bf[...], wob, preferred_element_type=jnp.float32)
        partial_f32[...] = part
        partial_bf[...] = part.astype(jnp.bfloat16)

        rs_descs = []
        for d in range(1, N_DEV):
            p = (my + d) % N_DEV
            rdma = pltpu.make_async_remote_copy(
                src_ref=partial_bf.at[pl.ds(p * CHUNK, CHUNK), :],
                dst_ref=rs_buf.at[d - 1],
                send_sem=rs_send.at[d - 1],
                recv_sem=rs_recv.at[d - 1],
                device_id=(p,),
                device_id_type=pl.DeviceIdType.MESH,
            )
            rdma.start()
            rs_descs.append(rdma)

        acc = partial_f32[pl.ds(my * CHUNK, CHUNK), :]
        for d in range(1, N_DEV):
            recv = pltpu.make_async_remote_copy(
                src_ref=rs_buf.at[d - 1], dst_ref=rs_buf.at[d - 1],
                send_sem=rs_send.at[d - 1], recv_sem=rs_recv.at[d - 1],
                device_id=(my,), device_id_type=pl.DeviceIdType.MESH,
            )
            recv.wait_recv()
            acc = acc + rs_buf[d - 1].astype(jnp.float32)

        for r in rs_descs:
            r.wait_send()

        ag_src[...] = acc.astype(jnp.bfloat16)
        out_ref[0, pl.ds(my * CHUNK, CHUNK), :] = acc

        ag_descs = []
        for d in range(1, N_DEV):
            p = (my + d) % N_DEV
            rdma = pltpu.make_async_remote_copy(
                src_ref=ag_src,
                dst_ref=ag_buf.at[d - 1],
                send_sem=ag_send.at[d - 1],
                recv_sem=ag_recv.at[d - 1],
                device_id=(p,),
                device_id_type=pl.DeviceIdType.MESH,
            )
            rdma.start()
            ag_descs.append(rdma)

        for d in range(1, N_DEV):
            recv = pltpu.make_async_remote_copy(
                src_ref=ag_buf.at[d - 1], dst_ref=ag_buf.at[d - 1],
                send_sem=ag_send.at[d - 1], recv_sem=ag_recv.at[d - 1],
                device_id=(my,), device_id_type=pl.DeviceIdType.MESH,
            )
            recv.wait_recv()
            src = (my - d) % N_DEV
            out_ref[0, pl.ds(src * CHUNK, CHUNK), :] = (
                ag_buf[d - 1].astype(jnp.float32))

        for r in ag_descs:
            r.wait_send()

    return pl.pallas_call(
        body,
        out_shape=jax.ShapeDtypeStruct((1, SQ, D), jnp.float32),
        in_specs=[
            pl.BlockSpec(memory_space=pltpu.VMEM),
            pl.BlockSpec(memory_space=pltpu.VMEM),
            pl.BlockSpec(memory_space=pl.ANY),
            pl.BlockSpec(memory_space=pl.ANY),
            pl.BlockSpec(memory_space=pltpu.VMEM),
        ],
        out_specs=pl.BlockSpec(memory_space=pltpu.VMEM),
        scratch_shapes=[
            pltpu.VMEM((2, SQ, HQ, DH), jnp.float32),
            pltpu.VMEM((SQ, HQ * DH), jnp.bfloat16),
            pltpu.VMEM((SQ, HQ * DH), jnp.bfloat16),
            pltpu.VMEM((SQ, D), jnp.float32),
            pltpu.VMEM((SQ, D), jnp.bfloat16),
            pltpu.VMEM((CHUNK, D), jnp.bfloat16),
            pltpu.VMEM((N_DEV - 1, CHUNK, D), jnp.bfloat16),
            pltpu.VMEM((N_DEV - 1, CHUNK, D), jnp.bfloat16),
            pltpu.SemaphoreType.DMA((2,)),
            pltpu.SemaphoreType.DMA((N_DEV - 1,)),
            pltpu.SemaphoreType.DMA((N_DEV - 1,)),
            pltpu.SemaphoreType.DMA((N_DEV - 1,)),
            pltpu.SemaphoreType.DMA((N_DEV - 1,)),
        ],
    )(x, Wq, K_ext, V_ext, Wo)
